# baseline (speedup 1.0000x reference)
import functools

import jax
import jax.numpy as jnp
from jax import lax
from jax.experimental import pallas as pl
from jax.experimental.pallas import tpu as pltpu
from jax.experimental.pallas import tpu_sc as plsc

N_NODES = 10000
N_EDGES = 160000
D_FEAT = 256

NC = 2
NS = 16
L = 16

DH = D_FEAT // NC
EPT = N_EDGES // NS
B = 80
NBLK = EPT // B
RPT = 624
RPT_LAST = N_NODES - (NS - 1) * RPT


def _body(atom_lo, atom_hi, bond, src_idx, dst_idx, out,
          acc, dst_v, src_v, gath_v, bond_v, gsem, bsem, ssem):
    c = lax.axis_index("c")
    s = lax.axis_index("s")
    r0 = s * RPT
    col = c * DH

    def init(n):
        @pl.when(jnp.logical_and(c == 0, (s == NS - 1) == (n == RPT_LAST)))
        def _():
            pltpu.sync_copy(atom_lo.at[pl.ds(r0, n)], acc.at[pl.ds(r0, n)])

        @pl.when(jnp.logical_and(c == 1, (s == NS - 1) == (n == RPT_LAST)))
        def _():
            pltpu.sync_copy(atom_hi.at[pl.ds(r0, n)], acc.at[pl.ds(r0, n)])

    init(RPT)
    init(RPT_LAST)

    plsc.subcore_barrier()

    e_base = s * EPT

    def start_loads(i):
        p = i % 2
        e0 = e_base + i * B
        pltpu.sync_copy(dst_idx.at[pl.ds(e0, B)], dst_v.at[p])
        pltpu.sync_copy(src_idx.at[pl.ds(e0, B)], src_v.at[p])

        @pl.when(c == 0)
        def _():
            pltpu.async_copy(atom_lo.at[dst_v.at[p]], gath_v.at[p], gsem)

        @pl.when(c == 1)
        def _():
            pltpu.async_copy(atom_hi.at[dst_v.at[p]], gath_v.at[p], gsem)

        pltpu.async_copy(bond.at[pl.ds(e0, B), pl.ds(col, DH)],
                         bond_v.at[p], bsem)

    start_loads(0)

    def block(i, carry):
        p = i % 2
        q = 1 - p
        pltpu.make_async_copy(atom_lo.at[dst_v.at[p]], gath_v.at[p],
                              gsem).wait()
        pltpu.make_async_copy(bond.at[pl.ds(0, B), pl.ds(col, DH)],
                              bond_v.at[p], bsem).wait()

        def row(b, carry2):
            for j in range(DH // L):
                sl = pl.ds(j * L, L)
                bond_v[p, b, sl] = bond_v[p, b, sl] * gath_v[p, b, sl]
            return carry2

        lax.fori_loop(0, B, row, 0)

        @pl.when(i >= 1)
        def _():
            pltpu.make_async_copy(bond_v.at[q], acc.at[src_v.at[q]],
                                  ssem).wait()

        pltpu.async_copy(bond_v.at[p], acc.at[src_v.at[p]], ssem, add=True)

        @pl.when(i + 1 < NBLK)
        def _():
            start_loads(i + 1)

        return carry

    lax.fori_loop(0, NBLK, block, 0)

    p_last = (NBLK - 1) % 2
    pltpu.make_async_copy(bond_v.at[p_last], acc.at[src_v.at[p_last]],
                          ssem).wait()

    plsc.subcore_barrier()

    def writeout(n):
        @pl.when((s == NS - 1) == (n == RPT_LAST))
        def _():
            pltpu.sync_copy(acc.at[pl.ds(r0, n)],
                            out.at[pl.ds(r0, n), pl.ds(col, DH)])

    writeout(RPT)
    writeout(RPT_LAST)


@jax.jit
def _run(atom_lo, atom_hi, bond, src_idx, dst_idx):
    mesh = plsc.VectorSubcoreMesh(core_axis_name="c", subcore_axis_name="s")
    return pl.kernel(
        _body,
        out_type=jax.ShapeDtypeStruct((N_NODES, D_FEAT), jnp.float32),
        mesh=mesh,
        scratch_types=[
            pltpu.VMEM_SHARED((N_NODES, DH), jnp.float32),
            pltpu.VMEM((2, B), jnp.int32),
            pltpu.VMEM((2, B), jnp.int32),
            pltpu.VMEM((2, B, DH), jnp.float32),
            pltpu.VMEM((2, B, DH), jnp.float32),
            pltpu.SemaphoreType.DMA,
            pltpu.SemaphoreType.DMA,
            pltpu.SemaphoreType.DMA,
        ],
    )(atom_lo, atom_hi, bond, src_idx, dst_idx)


def kernel(atom_features, bond_features, connectivity):
    atom_lo = atom_features[:, :DH]
    atom_hi = atom_features[:, DH:]
    src_idx = connectivity[:, 0].astype(jnp.int32)
    dst_idx = connectivity[:, 1].astype(jnp.int32)
    return _run(atom_lo, atom_hi, bond_features, src_idx, dst_idx)


# speedup vs baseline: 1.0002x; 1.0002x over previous
import functools

import jax
import jax.numpy as jnp
from jax import lax
from jax.experimental import pallas as pl
from jax.experimental.pallas import tpu as pltpu
from jax.experimental.pallas import tpu_sc as plsc

N_NODES = 10000
N_EDGES = 160000
D_FEAT = 256

NC = 2
NS = 16
L = 16

DH = D_FEAT // NC
EPT = N_EDGES // NS
B = 80
NBLK = EPT // B
RPT = 624
RPT_LAST = N_NODES - (NS - 1) * RPT


def _body(atom_lo, atom_hi, bond, src_idx, dst_idx, out,
          acc, dst_v, src_v, gath_v, bond_v, gsem, bsem, ssem):
    c = lax.axis_index("c")
    s = lax.axis_index("s")
    r0 = s * RPT
    col = c * DH

    def init(n):
        @pl.when(jnp.logical_and(c == 0, (s == NS - 1) == (n == RPT_LAST)))
        def _():
            pltpu.sync_copy(atom_lo.at[pl.ds(r0, n)], acc.at[pl.ds(r0, n)])

        @pl.when(jnp.logical_and(c == 1, (s == NS - 1) == (n == RPT_LAST)))
        def _():
            pltpu.sync_copy(atom_hi.at[pl.ds(r0, n)], acc.at[pl.ds(r0, n)])

    init(RPT)
    init(RPT_LAST)

    plsc.subcore_barrier()

    e_base = s * EPT

    def start_loads(i):
        p = i % 2
        e0 = e_base + i * B
        pltpu.sync_copy(dst_idx.at[pl.ds(e0, B)], dst_v.at[p])
        pltpu.sync_copy(src_idx.at[pl.ds(e0, B)], src_v.at[p])

        @pl.when(c == 0)
        def _():
            pltpu.async_copy(atom_lo.at[dst_v.at[p]], gath_v.at[p], gsem)

        @pl.when(c == 1)
        def _():
            pltpu.async_copy(atom_hi.at[dst_v.at[p]], gath_v.at[p], gsem)

        pltpu.async_copy(bond.at[pl.ds(e0, B), pl.ds(col, DH)],
                         bond_v.at[p], bsem)

    start_loads(0)

    def block(i, carry):
        p = i % 2
        q = 1 - p
        pltpu.make_async_copy(atom_lo.at[dst_v.at[p]], gath_v.at[p],
                              gsem).wait()
        pltpu.make_async_copy(bond.at[pl.ds(0, B), pl.ds(col, DH)],
                              bond_v.at[p], bsem).wait()

        def row(b, carry2):
            for j in range(DH // L):
                sl = pl.ds(j * L, L)
                bond_v[p, b, sl] = bond_v[p, b, sl] * gath_v[p, b, sl]
            return carry2

        lax.fori_loop(0, B, row, 0)

        @pl.when(i + 1 < NBLK)
        def _():
            start_loads(i + 1)

        pltpu.sync_copy(bond_v.at[p], acc.at[src_v.at[p]], add=True)
        return carry

    lax.fori_loop(0, NBLK, block, 0)

    plsc.subcore_barrier()

    def writeout(n):
        @pl.when((s == NS - 1) == (n == RPT_LAST))
        def _():
            pltpu.sync_copy(acc.at[pl.ds(r0, n)],
                            out.at[pl.ds(r0, n), pl.ds(col, DH)])

    writeout(RPT)
    writeout(RPT_LAST)


@jax.jit
def _run(atom_lo, atom_hi, bond, src_idx, dst_idx):
    mesh = plsc.VectorSubcoreMesh(core_axis_name="c", subcore_axis_name="s")
    return pl.kernel(
        _body,
        out_type=jax.ShapeDtypeStruct((N_NODES, D_FEAT), jnp.float32),
        mesh=mesh,
        scratch_types=[
            pltpu.VMEM_SHARED((N_NODES, DH), jnp.float32),
            pltpu.VMEM((2, B), jnp.int32),
            pltpu.VMEM((2, B), jnp.int32),
            pltpu.VMEM((2, B, DH), jnp.float32),
            pltpu.VMEM((2, B, DH), jnp.float32),
            pltpu.SemaphoreType.DMA,
            pltpu.SemaphoreType.DMA,
            pltpu.SemaphoreType.DMA,
        ],
    )(atom_lo, atom_hi, bond, src_idx, dst_idx)


def kernel(atom_features, bond_features, connectivity):
    atom_lo = atom_features[:, :DH]
    atom_hi = atom_features[:, DH:]
    src_idx = connectivity[:, 0].astype(jnp.int32)
    dst_idx = connectivity[:, 1].astype(jnp.int32)
    return _run(atom_lo, atom_hi, bond_features, src_idx, dst_idx)


# speedup vs baseline: 2.2720x; 2.2716x over previous
import functools

import jax
import jax.numpy as jnp
from jax import lax
from jax.experimental import pallas as pl
from jax.experimental.pallas import tpu as pltpu
from jax.experimental.pallas import tpu_sc as plsc

N_NODES = 10000
N_EDGES = 160000
D_FEAT = 256

NC = 2
NS = 16
L = 16

DH = D_FEAT // NC
EPT = N_EDGES // NS
B = 80
NBLK = EPT // B
RPT = 624
RPT_LAST = N_NODES - (NS - 1) * RPT


def _body(atom_lo, atom_hi, bond, src_idx, dst_idx, out,
          acc, dst_v0, dst_v1, src_v0, src_v1, gath_v0, gath_v1,
          bond_v0, bond_v1, gsem, bsem, ssem):
    c = lax.axis_index("c")
    s = lax.axis_index("s")
    r0 = s * RPT
    col = c * DH
    dst_b = (dst_v0, dst_v1)
    src_b = (src_v0, src_v1)
    gath_b = (gath_v0, gath_v1)
    bond_b = (bond_v0, bond_v1)

    def init(n):
        @pl.when(jnp.logical_and(c == 0, (s == NS - 1) == (n == RPT_LAST)))
        def _():
            pltpu.sync_copy(atom_lo.at[pl.ds(r0, n)], acc.at[pl.ds(r0, n)])

        @pl.when(jnp.logical_and(c == 1, (s == NS - 1) == (n == RPT_LAST)))
        def _():
            pltpu.sync_copy(atom_hi.at[pl.ds(r0, n)], acc.at[pl.ds(r0, n)])

    init(RPT)
    init(RPT_LAST)

    plsc.subcore_barrier()

    e_base = s * EPT

    def start_loads(p, ib):
        e0 = e_base + ib * B
        pltpu.sync_copy(dst_idx.at[pl.ds(e0, B)], dst_b[p])
        pltpu.sync_copy(src_idx.at[pl.ds(e0, B)], src_b[p])

        @pl.when(c == 0)
        def _():
            pltpu.async_copy(atom_lo.at[dst_b[p]], gath_b[p], gsem)

        @pl.when(c == 1)
        def _():
            pltpu.async_copy(atom_hi.at[dst_b[p]], gath_b[p], gsem)

        pltpu.async_copy(bond.at[pl.ds(e0, B), pl.ds(col, DH)],
                         bond_b[p], bsem)

    def wait_loads(p):
        pltpu.make_async_copy(atom_lo.at[dst_b[p]], gath_b[p], gsem).wait()
        pltpu.make_async_copy(bond.at[pl.ds(0, B), pl.ds(col, DH)],
                              bond_b[p], bsem).wait()

    def compute(p):
        gv = gath_b[p]
        bv = bond_b[p]

        def row(b, carry2):
            for j in range(DH // L):
                sl = pl.ds(j * L, L)
                bv[b, sl] = bv[b, sl] * gv[b, sl]
            return carry2

        lax.fori_loop(0, B, row, 0)

    def scatter_start(p):
        pltpu.async_copy(bond_b[p], acc.at[src_b[p]], ssem, add=True)

    def scatter_wait(p):
        pltpu.make_async_copy(bond_b[p], acc.at[src_b[p]], ssem).wait()

    start_loads(0, 0)

    def pair(k, carry):
        i0 = 2 * k

        @pl.when(k > 0)
        def _():
            scatter_wait(1)

        start_loads(1, i0 + 1)
        wait_loads(0)
        compute(0)
        scatter_start(0)
        wait_loads(1)
        compute(1)
        scatter_wait(0)
        start_loads(0, i0 + 2)
        scatter_start(1)
        return carry

    lax.fori_loop(0, (NBLK - 1) // 2, pair, 0)

    wait_loads(0)
    compute(0)
    scatter_wait(1)
    pltpu.sync_copy(bond_v0, acc.at[src_v0], add=True)

    plsc.subcore_barrier()

    def writeout(n):
        @pl.when((s == NS - 1) == (n == RPT_LAST))
        def _():
            pltpu.sync_copy(acc.at[pl.ds(r0, n)],
                            out.at[pl.ds(r0, n), pl.ds(col, DH)])

    writeout(RPT)
    writeout(RPT_LAST)


@jax.jit
def _run(atom_lo, atom_hi, bond, src_idx, dst_idx):
    mesh = plsc.VectorSubcoreMesh(core_axis_name="c", subcore_axis_name="s")
    return pl.kernel(
        _body,
        out_type=jax.ShapeDtypeStruct((N_NODES, D_FEAT), jnp.float32),
        mesh=mesh,
        scratch_types=[
            pltpu.VMEM_SHARED((N_NODES, DH), jnp.float32),
            pltpu.VMEM((B,), jnp.int32),
            pltpu.VMEM((B,), jnp.int32),
            pltpu.VMEM((B,), jnp.int32),
            pltpu.VMEM((B,), jnp.int32),
            pltpu.VMEM((B, DH), jnp.float32),
            pltpu.VMEM((B, DH), jnp.float32),
            pltpu.VMEM((B, DH), jnp.float32),
            pltpu.VMEM((B, DH), jnp.float32),
            pltpu.SemaphoreType.DMA,
            pltpu.SemaphoreType.DMA,
            pltpu.SemaphoreType.DMA,
        ],
    )(atom_lo, atom_hi, bond, src_idx, dst_idx)


def kernel(atom_features, bond_features, connectivity):
    atom_lo = atom_features[:, :DH]
    atom_hi = atom_features[:, DH:]
    src_idx = connectivity[:, 0].astype(jnp.int32)
    dst_idx = connectivity[:, 1].astype(jnp.int32)
    return _run(atom_lo, atom_hi, bond_features, src_idx, dst_idx)
